# baseline (speedup 1.0000x reference)
import jax
import jax.numpy as jnp
from jax.experimental import pallas as pl
from jax.experimental.pallas import tpu as pltpu

SEQ_BLK = 1024
NBUF = 4


def _add_kernel(x_ref, e_ref, o_hbm, obuf, sems):
    s = pl.program_id(0)
    b = pl.program_id(1)
    n_b = pl.num_programs(1)
    i = s * n_b + b
    slot = jax.lax.rem(i, NBUF)
    total = pl.num_programs(0) * n_b

    @pl.when(i >= NBUF)
    def _():
        pltpu.make_async_copy(obuf.at[slot], obuf.at[slot], sems.at[slot]).wait()

    obuf[slot] = x_ref[0] + e_ref[...]
    pltpu.make_async_copy(
        obuf.at[slot],
        o_hbm.at[b, pl.ds(s * SEQ_BLK, SEQ_BLK), :],
        sems.at[slot],
    ).start()

    @pl.when(i == total - 1)
    def _():
        for k in range(NBUF):
            pltpu.make_async_copy(obuf.at[k], obuf.at[k], sems.at[k]).wait()


def _kernel_tc(x, emb_table):
    B, S, D = x.shape
    grid = (S // SEQ_BLK, B)
    return pl.pallas_call(
        _add_kernel,
        grid=grid,
        in_specs=[
            pl.BlockSpec((1, SEQ_BLK, D), lambda s, b: (b, s, 0)),
            pl.BlockSpec((SEQ_BLK, D), lambda s, b: (s, 0)),
        ],
        out_specs=pl.BlockSpec(memory_space=pltpu.HBM),
        out_shape=jax.ShapeDtypeStruct((B, S, D), x.dtype),
        scratch_shapes=[
            pltpu.VMEM((NBUF, SEQ_BLK, D), jnp.float32),
            pltpu.SemaphoreType.DMA((NBUF,)),
        ],
        compiler_params=pltpu.CompilerParams(
            vmem_limit_bytes=100 * 1024 * 1024,
        ),
    )(x, emb_table)


def kernel(x, emb_table):
    if x.ndim == 2:
        return kernel(x[None], emb_table)[0]
    return _kernel_tc(x, emb_table)


# speedup vs baseline: 1.0519x; 1.0519x over previous
import jax
import jax.numpy as jnp
from jax.experimental import pallas as pl
from jax.experimental.pallas import tpu as pltpu

SEQ_BLK = 2048


def _add_kernel(x_ref, e_ref, o_ref):
    o_ref[0] = x_ref[0] + e_ref[...]


def _kernel_tc(x, emb_table):
    B, S, D = x.shape
    grid = (S // SEQ_BLK, B)
    return pl.pallas_call(
        _add_kernel,
        grid=grid,
        in_specs=[
            pl.BlockSpec((1, SEQ_BLK, D), lambda s, b: (b, s, 0)),
            pl.BlockSpec((SEQ_BLK, D), lambda s, b: (s, 0)),
        ],
        out_specs=pl.BlockSpec((1, SEQ_BLK, D), lambda s, b: (b, s, 0)),
        out_shape=jax.ShapeDtypeStruct((B, S, D), x.dtype),
        compiler_params=pltpu.CompilerParams(
            vmem_limit_bytes=100 * 1024 * 1024,
        ),
    )(x, emb_table)


def kernel(x, emb_table):
    if x.ndim == 2:
        return kernel(x[None], emb_table)[0]
    return _kernel_tc(x, emb_table)
